# baseline (device time: 144931 ns/iter reference)
import jax
import jax.numpy as jnp
from jax import lax
from jax.experimental import pallas as pl
from jax.experimental.pallas import tpu as pltpu

T = 4096
D = 1024
B = 256
BM = 256
NCH = T // BM
NC = T // B


def _body(xbf_ref, o_ref, s_ref, out_ref, xsf, rbuf, send_sems, recv_sems):
    my_x = lax.axis_index("x")
    my_y = lax.axis_index("y")
    my_z = lax.axis_index("z")
    peer = (my_x, 1 - my_y, my_z)

    S = s_ref[0]
    K = T - S
    sb = jnp.where(my_y == 0, K, 0)
    db = jnp.where(my_y == 0, 0, K)
    ns = (S + B - 1) // B

    barrier_sem = pltpu.get_barrier_semaphore()
    pl.semaphore_signal(
        barrier_sem, inc=1, device_id=peer,
        device_id_type=pl.DeviceIdType.MESH,
    )
    pl.semaphore_wait(barrier_sem, 1)

    c0 = sb // BM
    iota = lax.broadcasted_iota(jnp.int32, (BM, T), 1)
    rdmas = []
    for j in range(NCH + 1):
        if j < NCH:
            c = lax.rem(c0 + j, NCH)
            row0 = c * BM
            ids = o_ref[pl.ds(row0, BM), :]
            oh = (iota == ids).astype(jnp.bfloat16)
            res = lax.dot_general(
                oh, xbf_ref[...], (((1,), (0,)), ((), ())),
                preferred_element_type=jnp.float32,
            )
            xsf[pl.ds(row0 * D, BM * D)] = res.reshape(BM * D)

        i = j - 1
        if 0 <= i < NC:
            off = jnp.maximum(jnp.minimum(i * B, S - B), 0)
            rdma = pltpu.make_async_remote_copy(
                src_ref=xsf.at[pl.ds((sb + off) * D, B * D)],
                dst_ref=rbuf.at[pl.ds((db + off) * D, B * D)],
                send_sem=send_sems.at[i],
                recv_sem=recv_sems.at[i],
                device_id=peer,
                device_id_type=pl.DeviceIdType.MESH,
            )
            rdmas.append(rdma)

            @pl.when(i < ns)
            def _(rdma=rdma):
                rdma.start()

    rows = lax.broadcasted_iota(jnp.int32, (BM, 1), 0)

    def merge(c):
        rr = rows + c * BM
        mask = (rr >= sb) & (rr < sb + S)
        ks = xsf[pl.ds(c * BM * D, BM * D)].reshape(BM, D)
        rs = rbuf[pl.ds(c * BM * D, BM * D)].reshape(BM, D)
        out_ref[c * BM:(c + 1) * BM, :] = jnp.where(mask, rs, ks)

    touches = []
    for c in range(NCH):
        t = (c * BM < sb + S) & ((c + 1) * BM > sb)
        touches.append(t)

        @pl.when(jnp.logical_not(t))
        def _(c=c):
            merge(c)

    for i in range(NC):
        @pl.when(i < ns)
        def _(rdma=rdmas[i]):
            rdma.wait_send()
            rdma.wait_recv()

    for c in range(NCH):
        @pl.when(touches[c])
        def _(c=c):
            merge(c)


def kernel(x, dest):
    my_y = lax.axis_index("y")

    order = jnp.argsort(dest, stable=True).reshape(T, 1)

    ones = jnp.sum(dest)
    s = jnp.where(my_y == 0, ones, T - ones).astype(jnp.int32).reshape(1)

    return pl.pallas_call(
        _body,
        out_shape=jax.ShapeDtypeStruct((T, D), jnp.float32),
        in_specs=[
            pl.BlockSpec(memory_space=pltpu.VMEM),
            pl.BlockSpec(memory_space=pltpu.VMEM),
            pl.BlockSpec(memory_space=pltpu.SMEM),
        ],
        out_specs=pl.BlockSpec(memory_space=pltpu.VMEM),
        scratch_shapes=[
            pltpu.VMEM((T * D,), jnp.float32),
            pltpu.VMEM((T * D,), jnp.float32),
            pltpu.SemaphoreType.DMA((NC,)),
            pltpu.SemaphoreType.DMA((NC,)),
        ],
        compiler_params=pltpu.CompilerParams(
            collective_id=0, vmem_limit_bytes=100 * 1024 * 1024,
        ),
    )(x.astype(jnp.bfloat16), order, s)


# device time: 135120 ns/iter; 1.0726x vs baseline; 1.0726x over previous
import jax
import jax.numpy as jnp
from jax import lax
from jax.experimental import pallas as pl
from jax.experimental.pallas import tpu as pltpu

T = 4096
D = 1024
B = 256
BM = 256
NCH = T // BM
NC = T // B


def _body(x_ref, p_ref, s_ref, out_ref, xvc, xbf, xsf, rbuf,
          send_sems, recv_sems, xcp_sems):
    my_x = lax.axis_index("x")
    my_y = lax.axis_index("y")
    my_z = lax.axis_index("z")
    peer = (my_x, 1 - my_y, my_z)

    S = s_ref[0]
    K = T - S
    sb = jnp.where(my_y == 0, K, 0)
    db = jnp.where(my_y == 0, 0, K)
    ns = (S + B - 1) // B

    def xchunk_cp(c):
        return pltpu.make_async_copy(
            x_ref.at[pl.ds(c * BM, BM), :], xvc.at[c % 2],
            xcp_sems.at[c % 2],
        )

    xchunk_cp(0).start()
    for c in range(NCH):
        if c + 1 < NCH:
            xchunk_cp(c + 1).start()
        xchunk_cp(c).wait()
        xbf[c * BM:(c + 1) * BM, :] = xvc[c % 2].astype(jnp.bfloat16)

    barrier_sem = pltpu.get_barrier_semaphore()
    pl.semaphore_signal(
        barrier_sem, inc=1, device_id=peer,
        device_id_type=pl.DeviceIdType.MESH,
    )
    pl.semaphore_wait(barrier_sem, 1)

    c0 = sb // BM
    iota0 = lax.broadcasted_iota(jnp.int32, (BM, T), 0)
    pos = p_ref[...]
    rdmas = []
    for j in range(NCH + 1):
        if j < NCH:
            c = lax.rem(c0 + j, NCH)
            row0 = c * BM
            oh = (pos == iota0 + row0).astype(jnp.bfloat16)
            res = lax.dot_general(
                oh, xbf[...], (((1,), (0,)), ((), ())),
                preferred_element_type=jnp.float32,
            )
            xsf[pl.ds(row0 * D, BM * D)] = res.reshape(BM * D)

        i = j - 1
        if 0 <= i < NC:
            off = jnp.maximum(jnp.minimum(i * B, S - B), 0)
            rdma = pltpu.make_async_remote_copy(
                src_ref=xsf.at[pl.ds((sb + off) * D, B * D)],
                dst_ref=rbuf.at[pl.ds((db + off) * D, B * D)],
                send_sem=send_sems.at[i],
                recv_sem=recv_sems.at[i],
                device_id=peer,
                device_id_type=pl.DeviceIdType.MESH,
            )
            rdmas.append(rdma)

            @pl.when(i < ns)
            def _(rdma=rdma):
                rdma.start()

    rows = lax.broadcasted_iota(jnp.int32, (BM, 1), 0)

    def merge(c):
        rr = rows + c * BM
        mask = (rr >= sb) & (rr < sb + S)
        ks = xsf[pl.ds(c * BM * D, BM * D)].reshape(BM, D)
        rs = rbuf[pl.ds(c * BM * D, BM * D)].reshape(BM, D)
        out_ref[c * BM:(c + 1) * BM, :] = jnp.where(mask, rs, ks)

    touches = []
    for c in range(NCH):
        t = (c * BM < sb + S) & ((c + 1) * BM > sb)
        touches.append(t)

        @pl.when(jnp.logical_not(t))
        def _(c=c):
            merge(c)

    for i in range(NC):
        @pl.when(i < ns)
        def _(rdma=rdmas[i]):
            rdma.wait_send()
            rdma.wait_recv()

    for c in range(NCH):
        @pl.when(touches[c])
        def _(c=c):
            merge(c)


def kernel(x, dest):
    my_y = lax.axis_index("y")

    c1 = jnp.cumsum(dest)
    ones = c1[-1]
    iota = jnp.arange(T, dtype=dest.dtype)
    pos = jnp.where(dest == 0, iota - c1, (T - ones) + c1 - 1)
    pos = pos.astype(jnp.int32).reshape(1, T)

    s = jnp.where(my_y == 0, ones, T - ones).astype(jnp.int32).reshape(1)

    return pl.pallas_call(
        _body,
        out_shape=jax.ShapeDtypeStruct((T, D), jnp.float32),
        in_specs=[
            pl.BlockSpec(memory_space=pl.ANY),
            pl.BlockSpec(memory_space=pltpu.VMEM),
            pl.BlockSpec(memory_space=pltpu.SMEM),
        ],
        out_specs=pl.BlockSpec(memory_space=pltpu.VMEM),
        scratch_shapes=[
            pltpu.VMEM((2, BM, D), jnp.float32),
            pltpu.VMEM((T, D), jnp.bfloat16),
            pltpu.VMEM((T * D,), jnp.float32),
            pltpu.VMEM((T * D,), jnp.float32),
            pltpu.SemaphoreType.DMA((NC,)),
            pltpu.SemaphoreType.DMA((NC,)),
            pltpu.SemaphoreType.DMA((2,)),
        ],
        compiler_params=pltpu.CompilerParams(
            collective_id=0, vmem_limit_bytes=100 * 1024 * 1024,
        ),
    )(x, pos, s)
